# baseline (speedup 1.0000x reference)
import functools

import jax
import jax.numpy as jnp
from jax import lax
from jax.experimental import pallas as pl
from jax.experimental.pallas import tpu as pltpu
from jax.experimental.pallas import tpu_sc as plsc

B = 16384
D = 32
V_UID = 1000001
V_ZIP = 100001
GA_COLS = 2 * D
OUT_COLS = 4 * D + 1

NC = 2
NS = 16
NW = NC * NS
COLS_PER_W = B // NW
CHUNK = 128
NCHUNK = COLS_PER_W // CHUNK


def _sc_body(uo_hbm, zo_hbm, go_hbm, gi_hbm, mean_hbm, inv_hbm,
             uflat, zflat, gflat, out_hbm,
             ioff_v, gidx_v, vbuf, stat_v, sem):
  wid = lax.axis_index("s") * NC + lax.axis_index("c")

  pltpu.sync_copy(uo_hbm.at[wid], ioff_v.at[pl.ds(0, D)])
  pltpu.sync_copy(zo_hbm.at[wid], ioff_v.at[pl.ds(D, D)])
  pltpu.sync_copy(gi_hbm.at[pl.ds(wid * NCHUNK, NCHUNK)], gidx_v)
  pltpu.sync_copy(mean_hbm, stat_v.at[0])
  pltpu.sync_copy(inv_hbm, stat_v.at[1])

  cps = []
  for d in range(D):
    for j in range(NCHUNK):
      cps.append(pltpu.async_copy(
          uflat.at[ioff_v.at[d, j]],
          vbuf.at[d, pl.ds(j * CHUNK, CHUNK)], sem))
  for d in range(D):
    for j in range(NCHUNK):
      cps.append(pltpu.async_copy(
          zflat.at[ioff_v.at[D + d, j]],
          vbuf.at[D + d, pl.ds(j * CHUNK, CHUNK)], sem))

  mean = stat_v[0, :]
  inv = stat_v[1, :]
  seven = jnp.full((16,), 7, jnp.int32)
  for k in range(COLS_PER_W // 16):
    j, off = divmod(k * 16, CHUNK)
    a = lax.bitwise_and(gidx_v[j, pl.ds(off, 16)], seven)
    vbuf[4 * D, pl.ds(k * 16, 16)] = (a.astype(jnp.float32) - mean) * inv

  for cp in cps:
    cp.wait()

  pltpu.sync_copy(go_hbm.at[wid], ioff_v)
  cps = []
  for d in range(GA_COLS):
    for j in range(NCHUNK):
      cps.append(pltpu.async_copy(
          gflat.at[ioff_v.at[d, j]],
          vbuf.at[2 * D + d, pl.ds(j * CHUNK, CHUNK)], sem))
  for cp in cps:
    cp.wait()

  pltpu.sync_copy(vbuf, out_hbm.at[:, pl.ds(wid * COLS_PER_W, COLS_PER_W)])


@jax.jit
def _run(uo3, zo3, go3, gi2, mean16, inv16, uflat, zflat, gflat):
  mesh = plsc.VectorSubcoreMesh(core_axis_name="c", subcore_axis_name="s")
  out_t = pl.kernel(
      _sc_body,
      out_type=jax.ShapeDtypeStruct((OUT_COLS, B), jnp.float32),
      mesh=mesh,
      scratch_types=[
          pltpu.VMEM((GA_COLS, NCHUNK, CHUNK), jnp.int32),
          pltpu.VMEM((NCHUNK, CHUNK), jnp.int32),
          pltpu.VMEM((OUT_COLS, COLS_PER_W), jnp.float32),
          pltpu.VMEM((2, 16), jnp.float32),
          pltpu.SemaphoreType.DMA,
      ],
      compiler_params=pltpu.CompilerParams(use_tc_tiling_on_sc=False,
                                           needs_layout_passes=False),
  )(uo3, zo3, go3, gi2, mean16, inv16, uflat, zflat, gflat)
  return out_t.T


def kernel(user_id, user_zip_code, user_gender, bucketized_user_age,
           emb_user_id, emb_zip, emb_gender, emb_age, norm_mean, norm_var):
  n_age = emb_age.shape[0]
  n_gen = emb_gender.shape[0]
  emb_ga = jnp.concatenate([
      jnp.repeat(emb_gender, n_age, axis=0),
      jnp.tile(emb_age, (n_gen, 1)),
  ], axis=1)
  uflat = emb_user_id.T.reshape(-1)
  zflat = emb_zip.T.reshape(-1)
  gflat = emb_ga.T.reshape(-1)

  inv_std = 1.0 / jnp.sqrt(norm_var + 1e-6)
  mean16 = jnp.broadcast_to(norm_mean, (16,))
  inv16 = jnp.broadcast_to(inv_std, (16,))

  dims = jnp.arange(D, dtype=jnp.int32)[None, :, None, None]
  dims_ga = jnp.arange(GA_COLS, dtype=jnp.int32)[None, :, None, None]
  u4 = user_id.reshape(NW, 1, NCHUNK, CHUNK)
  z4 = user_zip_code.reshape(NW, 1, NCHUNK, CHUNK)
  ga_idx = user_gender * n_age + bucketized_user_age
  g4 = ga_idx.reshape(NW, 1, NCHUNK, CHUNK)
  uo3 = u4 + dims * V_UID
  zo3 = z4 + dims * V_ZIP
  go3 = g4 + dims_ga * (n_gen * n_age)
  gi2 = ga_idx.reshape(NW * NCHUNK, CHUNK)

  return _run(uo3, zo3, go3, gi2, mean16, inv16, uflat, zflat, gflat)


# speedup vs baseline: 7.6925x; 7.6925x over previous
import functools

import jax
import jax.numpy as jnp
from jax import lax
from jax.experimental import pallas as pl
from jax.experimental.pallas import tpu as pltpu
from jax.experimental.pallas import tpu_sc as plsc

B = 16384
D = 32
GA_COLS = 2 * D
OUT_COLS = 4 * D + 1

NC = 2
NS = 16
NW = NC * NS
ROWS_PER_W = B // NW
CHUNK = 128
NCHUNK = ROWS_PER_W // CHUNK


def _sc_body(uid_hbm, zip_hbm, ga_hbm, mean_hbm, inv_hbm,
             emb_uid, emb_zip, emb_ga, out_hbm,
             idx_v, ebuf, gabuf, cbuf, stat_v, sem, wsem):
  wid = lax.axis_index("s") * NC + lax.axis_index("c")
  base = wid * ROWS_PER_W

  ids = (uid_hbm, zip_hbm, ga_hbm)
  for f in range(3):
    pltpu.sync_copy(ids[f].at[pl.ds(wid * NCHUNK, NCHUNK)], idx_v.at[f])
  cps = []
  for f, (tab, dst) in enumerate(((emb_uid, ebuf.at[0]),
                                  (emb_zip, ebuf.at[1]),
                                  (emb_ga, gabuf))):
    for j in range(NCHUNK):
      cps.append(pltpu.async_copy(
          tab.at[idx_v.at[f, j]],
          dst.at[pl.ds(j * CHUNK, CHUNK)], sem))

  pltpu.sync_copy(mean_hbm, stat_v.at[0])
  pltpu.sync_copy(inv_hbm, stat_v.at[1])
  mean = stat_v[0, :]
  inv = stat_v[1, :]
  zero = jnp.zeros((16,), jnp.int32)
  seven = jnp.full((16,), 7, jnp.int32)
  for i in range(ROWS_PER_W // 16):
    j, off = divmod(i * 16, CHUNK)
    a = lax.bitwise_and(idx_v[2, j, pl.ds(off, 16)], seven)
    c = (a.astype(jnp.float32) - mean) * inv
    rows16 = lax.iota(jnp.int32, 16) + i * 16
    plsc.store_scatter(cbuf, [rows16, zero], c)

  for cp in cps:
    cp.wait()

  rows = pl.ds(base, ROWS_PER_W)
  wps = [
      pltpu.async_copy(ebuf.at[0], out_hbm.at[rows, pl.ds(0, D)], wsem),
      pltpu.async_copy(ebuf.at[1], out_hbm.at[rows, pl.ds(D, D)], wsem),
      pltpu.async_copy(gabuf, out_hbm.at[rows, pl.ds(2 * D, GA_COLS)], wsem),
      pltpu.async_copy(cbuf, out_hbm.at[rows, pl.ds(4 * D, 1)], wsem),
  ]
  for wp in wps:
    wp.wait()


@jax.jit
def _run(uid2, zip2, ga2, mean16, inv16, emb_uid, emb_zip, emb_ga):
  mesh = plsc.VectorSubcoreMesh(core_axis_name="c", subcore_axis_name="s")
  return pl.kernel(
      _sc_body,
      out_type=jax.ShapeDtypeStruct((B, OUT_COLS), jnp.float32),
      mesh=mesh,
      scratch_types=[
          pltpu.VMEM((3, NCHUNK, CHUNK), jnp.int32),
          pltpu.VMEM((2, ROWS_PER_W, D), jnp.float32),
          pltpu.VMEM((ROWS_PER_W, GA_COLS), jnp.float32),
          pltpu.VMEM((ROWS_PER_W, 1), jnp.float32),
          pltpu.VMEM((2, 16), jnp.float32),
          pltpu.SemaphoreType.DMA,
          pltpu.SemaphoreType.DMA,
      ],
      compiler_params=pltpu.CompilerParams(use_tc_tiling_on_sc=False,
                                           needs_layout_passes=False),
  )(uid2, zip2, ga2, mean16, inv16, emb_uid, emb_zip, emb_ga)


def kernel(user_id, user_zip_code, user_gender, bucketized_user_age,
           emb_user_id, emb_zip, emb_gender, emb_age, norm_mean, norm_var):
  inv_std = 1.0 / jnp.sqrt(norm_var + 1e-6)
  n_age = emb_age.shape[0]
  n_gen = emb_gender.shape[0]
  emb_ga = jnp.concatenate([
      jnp.repeat(emb_gender, n_age, axis=0),
      jnp.tile(emb_age, (n_gen, 1)),
  ], axis=1)
  ga_idx = user_gender * n_age + bucketized_user_age
  mean16 = jnp.broadcast_to(norm_mean, (16,))
  inv16 = jnp.broadcast_to(inv_std, (16,))
  shape2 = (NW * NCHUNK, CHUNK)
  return _run(user_id.reshape(shape2), user_zip_code.reshape(shape2),
              ga_idx.reshape(shape2), mean16, inv16,
              emb_user_id, emb_zip, emb_ga)
